# Initial kernel scaffold; baseline (speedup 1.0000x reference)
import functools

import jax
import jax.numpy as jnp
from jax import lax
from jax.experimental import pallas as pl
from jax.experimental.pallas import tpu as pltpu
from jax.experimental.pallas import tpu_sc as plsc

N_NODES = 10000
N_EDGES = 320000
IN_FEATS = 128
N_HIDDEN = 64

NC = 2
NS = 16
NW = NC * NS
CH = 80
NCHUNK = N_EDGES // NW // CH
NPAD = 10240
SROWS = NPAD // NS

_MESH = functools.partial(
    plsc.VectorSubcoreMesh, core_axis_name="c", subcore_axis_name="s",
    num_cores=NC, num_subcores=NS)

_Z16 = jnp.zeros((16,), jnp.float32)
_O16 = jnp.ones((16,), jnp.float32)


def _deg_body(src_hbm, dst_hbm, outdeg_hbm, indeg_hbm,
              idx_s, idx_d, ones_v, zbuf, degs_sh, degd_sh):
    c = lax.axis_index("c")
    s = lax.axis_index("s")
    wid = c * NS + s

    for k in range(CH // 16):
        ones_v[pl.ds(16 * k, 16)] = _O16

    def zrow(i, carry):
        zbuf[pl.ds(i * 16, 16)] = _Z16
        return carry
    lax.fori_loop(0, SROWS // 16, zrow, 0)

    pltpu.sync_copy(zbuf, degs_sh.at[pl.ds(s * SROWS, SROWS)])
    pltpu.sync_copy(zbuf, degd_sh.at[pl.ds(s * SROWS, SROWS)])
    plsc.subcore_barrier()

    pltpu.sync_copy(src_hbm.at[wid], idx_s)
    pltpu.sync_copy(dst_hbm.at[wid], idx_d)

    def body(j, carry):
        pltpu.sync_copy(ones_v, degs_sh.at[idx_s.at[j]], add=True)
        pltpu.sync_copy(ones_v, degd_sh.at[idx_d.at[j]], add=True)
        return carry
    lax.fori_loop(0, NCHUNK, body, 0)

    plsc.subcore_barrier()
    pltpu.sync_copy(degs_sh.at[pl.ds(s * SROWS, SROWS)], zbuf)
    pltpu.sync_copy(zbuf, outdeg_hbm.at[c, pl.ds(s * SROWS, SROWS)])
    pltpu.sync_copy(degd_sh.at[pl.ds(s * SROWS, SROWS)], zbuf)
    pltpu.sync_copy(zbuf, indeg_hbm.at[c, pl.ds(s * SROWS, SROWS)])


_deg_call = pl.kernel(
    _deg_body,
    out_type=(jax.ShapeDtypeStruct((NC, NPAD), jnp.float32),
              jax.ShapeDtypeStruct((NC, NPAD), jnp.float32)),
    mesh=_MESH(),
    scratch_types=[
        pltpu.VMEM((NCHUNK, CH), jnp.int32),
        pltpu.VMEM((NCHUNK, CH), jnp.int32),
        pltpu.VMEM((CH,), jnp.float32),
        pltpu.VMEM((SROWS,), jnp.float32),
        pltpu.VMEM_SHARED((NPAD,), jnp.float32),
        pltpu.VMEM_SHARED((NPAD,), jnp.float32),
    ],
)


def _msg_body(hw_hbm, src_hbm, dst_hbm, out_hbm,
              idx_s, idx_d, rows0, rows1, zbuf, acc_sh, sem0, sem1):
    c = lax.axis_index("c")
    s = lax.axis_index("s")
    wid = c * NS + s

    def zrow(i, carry):
        for q in range(4):
            zbuf[i, pl.ds(q * 16, 16)] = _Z16
        return carry
    lax.fori_loop(0, 64, zrow, 0)
    for t in range(SROWS // 64):
        pltpu.sync_copy(zbuf, acc_sh.at[pl.ds(s * SROWS + t * 64, 64)])
    plsc.subcore_barrier()

    pltpu.sync_copy(src_hbm.at[wid], idx_s)
    pltpu.sync_copy(dst_hbm.at[wid], idx_d)

    pltpu.async_copy(hw_hbm.at[idx_s.at[0]], rows0, sem0)

    def body(t, carry):
        j0 = 2 * t
        j1 = j0 + 1
        pltpu.async_copy(hw_hbm.at[idx_s.at[j1]], rows1, sem1)
        pltpu.make_async_copy(hw_hbm.at[idx_s.at[j0]], rows0, sem0).wait()
        pltpu.sync_copy(rows0, acc_sh.at[idx_d.at[j0]], add=True)

        @pl.when(j0 + 2 < NCHUNK)
        def _():
            pltpu.async_copy(hw_hbm.at[idx_s.at[j0 + 2]], rows0, sem0)

        pltpu.make_async_copy(hw_hbm.at[idx_s.at[j1]], rows1, sem1).wait()
        pltpu.sync_copy(rows1, acc_sh.at[idx_d.at[j1]], add=True)
        return carry
    lax.fori_loop(0, NCHUNK // 2, body, 0)

    pltpu.make_async_copy(hw_hbm.at[idx_s.at[NCHUNK - 1]], rows0, sem0).wait()
    pltpu.sync_copy(rows0, acc_sh.at[idx_d.at[NCHUNK - 1]], add=True)

    plsc.subcore_barrier()
    for t in range(SROWS // 64):
        pltpu.sync_copy(acc_sh.at[pl.ds(s * SROWS + t * 64, 64)], zbuf)
        pltpu.sync_copy(zbuf, out_hbm.at[c, pl.ds(s * SROWS + t * 64, 64)])


_msg_call = pl.kernel(
    _msg_body,
    out_type=jax.ShapeDtypeStruct((NC, NPAD, N_HIDDEN), jnp.float32),
    mesh=_MESH(),
    scratch_types=[
        pltpu.VMEM((NCHUNK, CH), jnp.int32),
        pltpu.VMEM((NCHUNK, CH), jnp.int32),
        pltpu.VMEM((CH, N_HIDDEN), jnp.float32),
        pltpu.VMEM((CH, N_HIDDEN), jnp.float32),
        pltpu.VMEM((64, N_HIDDEN), jnp.float32),
        pltpu.VMEM_SHARED((NPAD, N_HIDDEN), jnp.float32),
        pltpu.SemaphoreType.DMA,
        pltpu.SemaphoreType.DMA,
    ],
)


_RB = 1024


def _enc1_body(od0, od1, id0, id1, x_r, w1_r, ns_r, nd_r, hw1_r):
    ns = lax.rsqrt(jnp.maximum(od0[...] + od1[...], 1.0))
    nd = lax.rsqrt(jnp.maximum(id0[...] + id1[...], 1.0))
    ns_r[...] = ns
    nd_r[...] = nd
    hw1_r[...] = jnp.dot(x_r[...] * ns[:, None], w1_r[...],
                         preferred_element_type=jnp.float32)


def _enc1_call(od0, od1, id0, id1, x, w1):
    grid = (NPAD // _RB,)
    return pl.pallas_call(
        _enc1_body,
        grid=grid,
        in_specs=[
            pl.BlockSpec((_RB,), lambda i: (i,)),
            pl.BlockSpec((_RB,), lambda i: (i,)),
            pl.BlockSpec((_RB,), lambda i: (i,)),
            pl.BlockSpec((_RB,), lambda i: (i,)),
            pl.BlockSpec((_RB, IN_FEATS), lambda i: (i, 0)),
            pl.BlockSpec((IN_FEATS, N_HIDDEN), lambda i: (0, 0)),
        ],
        out_specs=[
            pl.BlockSpec((_RB,), lambda i: (i,)),
            pl.BlockSpec((_RB,), lambda i: (i,)),
            pl.BlockSpec((_RB, N_HIDDEN), lambda i: (i, 0)),
        ],
        out_shape=[
            jax.ShapeDtypeStruct((NPAD,), jnp.float32),
            jax.ShapeDtypeStruct((NPAD,), jnp.float32),
            jax.ShapeDtypeStruct((N_NODES, N_HIDDEN), jnp.float32),
        ],
    )(od0, od1, id0, id1, x, w1)


def _enc2_body(a0, a1, nd_r, ns_r, b1_r, w2_r, hw2_r):
    agg = (a0[...] + a1[...]) * nd_r[...][:, None] + b1_r[...]
    h1 = jnp.maximum(agg, 0.0)
    hw2_r[...] = jnp.dot(h1 * ns_r[...][:, None], w2_r[...],
                         preferred_element_type=jnp.float32)


def _enc2_call(a0, a1, nd, ns, b1, w2):
    grid = (NPAD // _RB,)
    return pl.pallas_call(
        _enc2_body,
        grid=grid,
        in_specs=[
            pl.BlockSpec((_RB, N_HIDDEN), lambda i: (i, 0)),
            pl.BlockSpec((_RB, N_HIDDEN), lambda i: (i, 0)),
            pl.BlockSpec((_RB,), lambda i: (i,)),
            pl.BlockSpec((_RB,), lambda i: (i,)),
            pl.BlockSpec((1, N_HIDDEN), lambda i: (0, 0)),
            pl.BlockSpec((N_HIDDEN, N_HIDDEN), lambda i: (0, 0)),
        ],
        out_specs=pl.BlockSpec((_RB, N_HIDDEN), lambda i: (i, 0)),
        out_shape=jax.ShapeDtypeStruct((N_NODES, N_HIDDEN), jnp.float32),
    )(a0, a1, nd, ns, b1, w2)


_DR = 512
_DC = 2048


def _dec_body(ar0, ar1, ac0, ac1, ndr, ndc, b2_r, out_r):
    h2r = (ar0[...] + ar1[...]) * ndr[...][:, None] + b2_r[...]
    h2c = (ac0[...] + ac1[...]) * ndc[...][:, None] + b2_r[...]
    z = lax.dot_general(h2r, h2c, (((1,), (1,)), ((), ())),
                        preferred_element_type=jnp.float32)
    out_r[...] = jax.nn.sigmoid(z)


def _dec_call(a0, a1, nd, b2):
    grid = (NPAD // _DR, NPAD // _DC)
    return pl.pallas_call(
        _dec_body,
        grid=grid,
        in_specs=[
            pl.BlockSpec((_DR, N_HIDDEN), lambda i, j: (i, 0)),
            pl.BlockSpec((_DR, N_HIDDEN), lambda i, j: (i, 0)),
            pl.BlockSpec((_DC, N_HIDDEN), lambda i, j: (j, 0)),
            pl.BlockSpec((_DC, N_HIDDEN), lambda i, j: (j, 0)),
            pl.BlockSpec((_DR,), lambda i, j: (i,)),
            pl.BlockSpec((_DC,), lambda i, j: (j,)),
            pl.BlockSpec((1, N_HIDDEN), lambda i, j: (0, 0)),
        ],
        out_specs=pl.BlockSpec((_DR, _DC), lambda i, j: (i, j)),
        out_shape=jax.ShapeDtypeStruct((N_NODES, N_NODES), jnp.float32),
        compiler_params=pltpu.CompilerParams(
            dimension_semantics=("parallel", "parallel")),
    )(a0, a1, a0, a1, nd, nd, b2)


def kernel(x, edge_index, W1, b1, W2, b2):
    src = edge_index[0].astype(jnp.int32)
    dst = edge_index[1].astype(jnp.int32)
    src3 = src.reshape(NW, NCHUNK, CH)
    dst3 = dst.reshape(NW, NCHUNK, CH)

    outdeg_p, indeg_p = _deg_call(src3, dst3)
    ns, nd, hw1 = _enc1_call(outdeg_p[0], outdeg_p[1],
                             indeg_p[0], indeg_p[1], x, W1)
    agg1 = _msg_call(hw1, src3, dst3)
    hw2 = _enc2_call(agg1[0], agg1[1], nd, ns, b1.reshape(1, N_HIDDEN), W2)
    agg2 = _msg_call(hw2, src3, dst3)
    adj = _dec_call(agg2[0], agg2[1], nd, b2.reshape(1, N_HIDDEN))
    return adj


# speedup vs baseline: 7.5140x; 7.5140x over previous
import functools

import jax
import jax.numpy as jnp
from jax import lax
from jax.experimental import pallas as pl
from jax.experimental.pallas import tpu as pltpu
from jax.experimental.pallas import tpu_sc as plsc

N_NODES = 10000
N_EDGES = 320000
IN_FEATS = 128
N_HIDDEN = 64

NC = 2
NS = 16
NW = NC * NS
CH = 80
NCHUNK = N_EDGES // NW // CH
NPAD = 10240
SROWS = NPAD // NS

_MESH = functools.partial(
    plsc.VectorSubcoreMesh, core_axis_name="c", subcore_axis_name="s",
    num_cores=NC, num_subcores=NS)

def _deg_body(src_hbm, dst_hbm, outdeg_hbm, indeg_hbm,
              idx_s, idx_d, ones_v, zbuf, degs_sh, degd_sh):
    c = lax.axis_index("c")
    s = lax.axis_index("s")
    wid = c * NS + s
    _Z16 = jnp.zeros((16,), jnp.float32)
    _O16 = jnp.ones((16,), jnp.float32)

    for k in range(CH // 16):
        ones_v[pl.ds(16 * k, 16)] = _O16

    def zrow(i, carry):
        zbuf[pl.ds(i * 16, 16)] = _Z16
        return carry
    lax.fori_loop(0, SROWS // 16, zrow, 0)

    pltpu.sync_copy(zbuf, degs_sh.at[pl.ds(s * SROWS, SROWS)])
    pltpu.sync_copy(zbuf, degd_sh.at[pl.ds(s * SROWS, SROWS)])
    plsc.subcore_barrier()

    pltpu.sync_copy(src_hbm.at[wid], idx_s)
    pltpu.sync_copy(dst_hbm.at[wid], idx_d)

    def body(j, carry):
        pltpu.sync_copy(ones_v, degs_sh.at[idx_s.at[j]], add=True)
        pltpu.sync_copy(ones_v, degd_sh.at[idx_d.at[j]], add=True)
        return carry
    lax.fori_loop(0, NCHUNK, body, 0)

    plsc.subcore_barrier()
    pltpu.sync_copy(degs_sh.at[pl.ds(s * SROWS, SROWS)], zbuf)
    pltpu.sync_copy(zbuf, outdeg_hbm.at[c, pl.ds(s * SROWS, SROWS)])
    pltpu.sync_copy(degd_sh.at[pl.ds(s * SROWS, SROWS)], zbuf)
    pltpu.sync_copy(zbuf, indeg_hbm.at[c, pl.ds(s * SROWS, SROWS)])


_deg_call = pl.kernel(
    _deg_body,
    out_type=(jax.ShapeDtypeStruct((NC, NPAD), jnp.float32),
              jax.ShapeDtypeStruct((NC, NPAD), jnp.float32)),
    mesh=_MESH(),
    scratch_types=[
        pltpu.VMEM((NCHUNK, CH), jnp.int32),
        pltpu.VMEM((NCHUNK, CH), jnp.int32),
        pltpu.VMEM((CH,), jnp.float32),
        pltpu.VMEM((SROWS,), jnp.float32),
        pltpu.VMEM_SHARED((NPAD,), jnp.float32),
        pltpu.VMEM_SHARED((NPAD,), jnp.float32),
    ],
    compiler_params=pltpu.CompilerParams(use_tc_tiling_on_sc=False),
)


def _msg_body(hw_hbm, src_hbm, dst_hbm, out_hbm,
              idx_s, idx_d, rows0, rows1, zbuf, acc_sh, sem0, sem1):
    c = lax.axis_index("c")
    s = lax.axis_index("s")
    wid = c * NS + s
    _Z16 = jnp.zeros((16,), jnp.float32)

    def zrow(i, carry):
        for q in range(4):
            zbuf[i, pl.ds(q * 16, 16)] = _Z16
        return carry
    lax.fori_loop(0, 64, zrow, 0)
    for t in range(SROWS // 64):
        pltpu.sync_copy(zbuf, acc_sh.at[pl.ds(s * SROWS + t * 64, 64)])
    plsc.subcore_barrier()

    pltpu.sync_copy(src_hbm.at[wid], idx_s)
    pltpu.sync_copy(dst_hbm.at[wid], idx_d)

    pltpu.async_copy(hw_hbm.at[idx_s.at[0]], rows0, sem0)

    def body(t, carry):
        j0 = 2 * t
        j1 = j0 + 1
        pltpu.async_copy(hw_hbm.at[idx_s.at[j1]], rows1, sem1)
        pltpu.make_async_copy(hw_hbm.at[idx_s.at[j0]], rows0, sem0).wait()
        pltpu.sync_copy(rows0, acc_sh.at[idx_d.at[j0]], add=True)

        @pl.when(j0 + 2 < NCHUNK)
        def _():
            pltpu.async_copy(hw_hbm.at[idx_s.at[j0 + 2]], rows0, sem0)

        pltpu.make_async_copy(hw_hbm.at[idx_s.at[j1]], rows1, sem1).wait()
        pltpu.sync_copy(rows1, acc_sh.at[idx_d.at[j1]], add=True)
        return carry
    lax.fori_loop(0, NCHUNK // 2, body, 0)

    pltpu.make_async_copy(hw_hbm.at[idx_s.at[NCHUNK - 1]], rows0, sem0).wait()
    pltpu.sync_copy(rows0, acc_sh.at[idx_d.at[NCHUNK - 1]], add=True)

    plsc.subcore_barrier()
    for t in range(SROWS // 64):
        pltpu.sync_copy(acc_sh.at[pl.ds(s * SROWS + t * 64, 64)], zbuf)
        pltpu.sync_copy(zbuf, out_hbm.at[c, pl.ds(s * SROWS + t * 64, 64)])


_msg_call = pl.kernel(
    _msg_body,
    out_type=jax.ShapeDtypeStruct((NC, NPAD, N_HIDDEN), jnp.float32),
    mesh=_MESH(),
    scratch_types=[
        pltpu.VMEM((NCHUNK, CH), jnp.int32),
        pltpu.VMEM((NCHUNK, CH), jnp.int32),
        pltpu.VMEM((CH, N_HIDDEN), jnp.float32),
        pltpu.VMEM((CH, N_HIDDEN), jnp.float32),
        pltpu.VMEM((64, N_HIDDEN), jnp.float32),
        pltpu.VMEM_SHARED((NPAD, N_HIDDEN), jnp.float32),
        pltpu.SemaphoreType.DMA,
        pltpu.SemaphoreType.DMA,
    ],
    compiler_params=pltpu.CompilerParams(use_tc_tiling_on_sc=False),
)


_RB = 1024


def _enc1_body(od0, od1, id0, id1, x_r, w1_r, ns_r, nd_r, hw1_r):
    ns = lax.rsqrt(jnp.maximum(od0[...] + od1[...], 1.0))
    nd = lax.rsqrt(jnp.maximum(id0[...] + id1[...], 1.0))
    ns_r[...] = ns
    nd_r[...] = nd
    hw1_r[...] = jnp.dot(x_r[...] * ns[:, None], w1_r[...],
                         preferred_element_type=jnp.float32)


def _enc1_call(od0, od1, id0, id1, x, w1):
    grid = (NPAD // _RB,)
    return pl.pallas_call(
        _enc1_body,
        grid=grid,
        in_specs=[
            pl.BlockSpec((_RB,), lambda i: (i,)),
            pl.BlockSpec((_RB,), lambda i: (i,)),
            pl.BlockSpec((_RB,), lambda i: (i,)),
            pl.BlockSpec((_RB,), lambda i: (i,)),
            pl.BlockSpec((_RB, IN_FEATS), lambda i: (i, 0)),
            pl.BlockSpec((IN_FEATS, N_HIDDEN), lambda i: (0, 0)),
        ],
        out_specs=[
            pl.BlockSpec((_RB,), lambda i: (i,)),
            pl.BlockSpec((_RB,), lambda i: (i,)),
            pl.BlockSpec((_RB, N_HIDDEN), lambda i: (i, 0)),
        ],
        out_shape=[
            jax.ShapeDtypeStruct((NPAD,), jnp.float32),
            jax.ShapeDtypeStruct((NPAD,), jnp.float32),
            jax.ShapeDtypeStruct((N_NODES, N_HIDDEN), jnp.float32),
        ],
    )(od0, od1, id0, id1, x, w1)


def _enc2_body(a0, a1, nd_r, ns_r, b1_r, w2_r, hw2_r):
    agg = (a0[...] + a1[...]) * nd_r[...][:, None] + b1_r[...]
    h1 = jnp.maximum(agg, 0.0)
    hw2_r[...] = jnp.dot(h1 * ns_r[...][:, None], w2_r[...],
                         preferred_element_type=jnp.float32)


def _enc2_call(a0, a1, nd, ns, b1, w2):
    grid = (NPAD // _RB,)
    return pl.pallas_call(
        _enc2_body,
        grid=grid,
        in_specs=[
            pl.BlockSpec((_RB, N_HIDDEN), lambda i: (i, 0)),
            pl.BlockSpec((_RB, N_HIDDEN), lambda i: (i, 0)),
            pl.BlockSpec((_RB,), lambda i: (i,)),
            pl.BlockSpec((_RB,), lambda i: (i,)),
            pl.BlockSpec((1, N_HIDDEN), lambda i: (0, 0)),
            pl.BlockSpec((N_HIDDEN, N_HIDDEN), lambda i: (0, 0)),
        ],
        out_specs=pl.BlockSpec((_RB, N_HIDDEN), lambda i: (i, 0)),
        out_shape=jax.ShapeDtypeStruct((N_NODES, N_HIDDEN), jnp.float32),
    )(a0, a1, nd, ns, b1, w2)


_DR = 512
_DC = 2048


def _dec_body(ar0, ar1, ac0, ac1, ndr, ndc, b2_r, out_r):
    h2r = (ar0[...] + ar1[...]) * ndr[...][:, None] + b2_r[...]
    h2c = (ac0[...] + ac1[...]) * ndc[...][:, None] + b2_r[...]
    z = lax.dot_general(h2r, h2c, (((1,), (1,)), ((), ())),
                        preferred_element_type=jnp.float32)
    out_r[...] = jax.nn.sigmoid(z)


def _dec_call(a0, a1, nd, b2):
    grid = (NPAD // _DR, NPAD // _DC)
    return pl.pallas_call(
        _dec_body,
        grid=grid,
        in_specs=[
            pl.BlockSpec((_DR, N_HIDDEN), lambda i, j: (i, 0)),
            pl.BlockSpec((_DR, N_HIDDEN), lambda i, j: (i, 0)),
            pl.BlockSpec((_DC, N_HIDDEN), lambda i, j: (j, 0)),
            pl.BlockSpec((_DC, N_HIDDEN), lambda i, j: (j, 0)),
            pl.BlockSpec((_DR,), lambda i, j: (i,)),
            pl.BlockSpec((_DC,), lambda i, j: (j,)),
            pl.BlockSpec((1, N_HIDDEN), lambda i, j: (0, 0)),
        ],
        out_specs=pl.BlockSpec((_DR, _DC), lambda i, j: (i, j)),
        out_shape=jax.ShapeDtypeStruct((N_NODES, N_NODES), jnp.float32),
        compiler_params=pltpu.CompilerParams(
            dimension_semantics=("parallel", "parallel")),
    )(a0, a1, a0, a1, nd, nd, b2)


def kernel(x, edge_index, W1, b1, W2, b2):
    src = edge_index[0].astype(jnp.int32)
    dst = edge_index[1].astype(jnp.int32)
    src3 = src.reshape(NW, NCHUNK, CH)
    dst3 = dst.reshape(NW, NCHUNK, CH)

    outdeg_p, indeg_p = _deg_call(src3, dst3)
    ns, nd, hw1 = _enc1_call(outdeg_p[0], outdeg_p[1],
                             indeg_p[0], indeg_p[1], x, W1)
    agg1 = _msg_call(hw1, src3, dst3)
    hw2 = _enc2_call(agg1[0], agg1[1], nd, ns, b1.reshape(1, N_HIDDEN), W2)
    agg2 = _msg_call(hw2, src3, dst3)
    adj = _dec_call(agg2[0], agg2[1], nd, b2.reshape(1, N_HIDDEN))
    return adj


# speedup vs baseline: 8.4412x; 1.1234x over previous
import functools

import jax
import jax.numpy as jnp
from jax import lax
from jax.experimental import pallas as pl
from jax.experimental.pallas import tpu as pltpu
from jax.experimental.pallas import tpu_sc as plsc

N_NODES = 10000
N_EDGES = 320000
IN_FEATS = 128
N_HIDDEN = 64

NC = 2
NS = 16
NW = NC * NS
CH = 80
NCHUNK = N_EDGES // NW // CH
MCH = 100
MCHUNK = N_EDGES // NW // MCH
NBUF = 4
NPAD = 10240
SROWS = NPAD // NS

_MESH = functools.partial(
    plsc.VectorSubcoreMesh, core_axis_name="c", subcore_axis_name="s",
    num_cores=NC, num_subcores=NS)

def _deg_body(src_hbm, dst_hbm, outdeg_hbm, indeg_hbm,
              idx_s, idx_d, ones_v, zbuf, degs_sh, degd_sh):
    c = lax.axis_index("c")
    s = lax.axis_index("s")
    wid = c * NS + s
    _Z16 = jnp.zeros((16,), jnp.float32)
    _O16 = jnp.ones((16,), jnp.float32)

    for k in range(CH // 16):
        ones_v[pl.ds(16 * k, 16)] = _O16

    def zrow(i, carry):
        zbuf[pl.ds(i * 16, 16)] = _Z16
        return carry
    lax.fori_loop(0, SROWS // 16, zrow, 0)

    pltpu.sync_copy(zbuf, degs_sh.at[pl.ds(s * SROWS, SROWS)])
    pltpu.sync_copy(zbuf, degd_sh.at[pl.ds(s * SROWS, SROWS)])
    plsc.subcore_barrier()

    pltpu.sync_copy(src_hbm.at[wid], idx_s)
    pltpu.sync_copy(dst_hbm.at[wid], idx_d)

    def body(j, carry):
        pltpu.sync_copy(ones_v, degs_sh.at[idx_s.at[j]], add=True)
        pltpu.sync_copy(ones_v, degd_sh.at[idx_d.at[j]], add=True)
        return carry
    lax.fori_loop(0, NCHUNK, body, 0)

    plsc.subcore_barrier()
    pltpu.sync_copy(degs_sh.at[pl.ds(s * SROWS, SROWS)], zbuf)
    pltpu.sync_copy(zbuf, outdeg_hbm.at[c, pl.ds(s * SROWS, SROWS)])
    pltpu.sync_copy(degd_sh.at[pl.ds(s * SROWS, SROWS)], zbuf)
    pltpu.sync_copy(zbuf, indeg_hbm.at[c, pl.ds(s * SROWS, SROWS)])


_deg_call = pl.kernel(
    _deg_body,
    out_type=(jax.ShapeDtypeStruct((NC, NPAD), jnp.float32),
              jax.ShapeDtypeStruct((NC, NPAD), jnp.float32)),
    mesh=_MESH(),
    scratch_types=[
        pltpu.VMEM((NCHUNK, CH), jnp.int32),
        pltpu.VMEM((NCHUNK, CH), jnp.int32),
        pltpu.VMEM((CH,), jnp.float32),
        pltpu.VMEM((SROWS,), jnp.float32),
        pltpu.VMEM_SHARED((NPAD,), jnp.float32),
        pltpu.VMEM_SHARED((NPAD,), jnp.float32),
    ],
    compiler_params=pltpu.CompilerParams(use_tc_tiling_on_sc=False),
)


def _msg_body(hw_hbm, src_hbm, dst_hbm, out_hbm,
              idx_s, idx_d, rows, zbuf, acc_sh, gsem, ssem):
    c = lax.axis_index("c")
    s = lax.axis_index("s")
    wid = c * NS + s
    _Z16 = jnp.zeros((16,), jnp.float32)

    def zrow(i, carry):
        for q in range(4):
            zbuf[i, pl.ds(q * 16, 16)] = _Z16
        return carry
    lax.fori_loop(0, 64, zrow, 0)
    for t in range(SROWS // 64):
        pltpu.sync_copy(zbuf, acc_sh.at[pl.ds(s * SROWS + t * 64, 64)])
    plsc.subcore_barrier()

    pltpu.sync_copy(src_hbm.at[wid], idx_s)
    pltpu.sync_copy(dst_hbm.at[wid], idx_d)

    for b in range(NBUF):
        pltpu.async_copy(hw_hbm.at[idx_s.at[b]], rows.at[b], gsem.at[b])

    def body(t, carry):
        j0 = NBUF * t
        for b in range(NBUF):
            pltpu.make_async_copy(hw_hbm.at[idx_s.at[j0 + b]],
                                  rows.at[b], gsem.at[b]).wait()
            pltpu.async_copy(rows.at[b], acc_sh.at[idx_d.at[j0 + b]],
                             ssem.at[b], add=True)
        for b in range(NBUF):
            @pl.when(j0 + b + NBUF < MCHUNK)
            def _():
                pltpu.make_async_copy(rows.at[b],
                                      acc_sh.at[idx_d.at[j0 + b]],
                                      ssem.at[b]).wait()
                pltpu.async_copy(hw_hbm.at[idx_s.at[j0 + b + NBUF]],
                                 rows.at[b], gsem.at[b])
        return carry
    lax.fori_loop(0, MCHUNK // NBUF, body, 0)

    for b in range(NBUF):
        pltpu.make_async_copy(rows.at[b],
                              acc_sh.at[idx_d.at[MCHUNK - NBUF + b]],
                              ssem.at[b]).wait()

    plsc.subcore_barrier()
    for t in range(SROWS // 64):
        pltpu.sync_copy(acc_sh.at[pl.ds(s * SROWS + t * 64, 64)], zbuf)
        pltpu.sync_copy(zbuf, out_hbm.at[c, pl.ds(s * SROWS + t * 64, 64)])


_msg_call = pl.kernel(
    _msg_body,
    out_type=jax.ShapeDtypeStruct((NC, NPAD, N_HIDDEN), jnp.float32),
    mesh=_MESH(),
    scratch_types=[
        pltpu.VMEM((MCHUNK, MCH), jnp.int32),
        pltpu.VMEM((MCHUNK, MCH), jnp.int32),
        pltpu.VMEM((NBUF, MCH, N_HIDDEN), jnp.float32),
        pltpu.VMEM((64, N_HIDDEN), jnp.float32),
        pltpu.VMEM_SHARED((NPAD, N_HIDDEN), jnp.float32),
        pltpu.SemaphoreType.DMA((NBUF,)),
        pltpu.SemaphoreType.DMA((NBUF,)),
    ],
    compiler_params=pltpu.CompilerParams(use_tc_tiling_on_sc=False),
)


_RB = 1024


def _enc1_body(od0, od1, id0, id1, x_r, w1_r, ns_r, nd_r, hw1_r):
    ns = lax.rsqrt(jnp.maximum(od0[...] + od1[...], 1.0))
    nd = lax.rsqrt(jnp.maximum(id0[...] + id1[...], 1.0))
    ns_r[...] = ns
    nd_r[...] = nd
    hw1_r[...] = jnp.dot(x_r[...] * ns[:, None], w1_r[...],
                         preferred_element_type=jnp.float32)


def _enc1_call(od0, od1, id0, id1, x, w1):
    grid = (NPAD // _RB,)
    return pl.pallas_call(
        _enc1_body,
        grid=grid,
        in_specs=[
            pl.BlockSpec((_RB,), lambda i: (i,)),
            pl.BlockSpec((_RB,), lambda i: (i,)),
            pl.BlockSpec((_RB,), lambda i: (i,)),
            pl.BlockSpec((_RB,), lambda i: (i,)),
            pl.BlockSpec((_RB, IN_FEATS), lambda i: (i, 0)),
            pl.BlockSpec((IN_FEATS, N_HIDDEN), lambda i: (0, 0)),
        ],
        out_specs=[
            pl.BlockSpec((_RB,), lambda i: (i,)),
            pl.BlockSpec((_RB,), lambda i: (i,)),
            pl.BlockSpec((_RB, N_HIDDEN), lambda i: (i, 0)),
        ],
        out_shape=[
            jax.ShapeDtypeStruct((NPAD,), jnp.float32),
            jax.ShapeDtypeStruct((NPAD,), jnp.float32),
            jax.ShapeDtypeStruct((N_NODES, N_HIDDEN), jnp.float32),
        ],
    )(od0, od1, id0, id1, x, w1)


def _enc2_body(a0, a1, nd_r, ns_r, b1_r, w2_r, hw2_r):
    agg = (a0[...] + a1[...]) * nd_r[...][:, None] + b1_r[...]
    h1 = jnp.maximum(agg, 0.0)
    hw2_r[...] = jnp.dot(h1 * ns_r[...][:, None], w2_r[...],
                         preferred_element_type=jnp.float32)


def _enc2_call(a0, a1, nd, ns, b1, w2):
    grid = (NPAD // _RB,)
    return pl.pallas_call(
        _enc2_body,
        grid=grid,
        in_specs=[
            pl.BlockSpec((_RB, N_HIDDEN), lambda i: (i, 0)),
            pl.BlockSpec((_RB, N_HIDDEN), lambda i: (i, 0)),
            pl.BlockSpec((_RB,), lambda i: (i,)),
            pl.BlockSpec((_RB,), lambda i: (i,)),
            pl.BlockSpec((1, N_HIDDEN), lambda i: (0, 0)),
            pl.BlockSpec((N_HIDDEN, N_HIDDEN), lambda i: (0, 0)),
        ],
        out_specs=pl.BlockSpec((_RB, N_HIDDEN), lambda i: (i, 0)),
        out_shape=jax.ShapeDtypeStruct((N_NODES, N_HIDDEN), jnp.float32),
    )(a0, a1, nd, ns, b1, w2)


_DR = 512
_DC = 2048


def _dec_body(ar0, ar1, ac0, ac1, ndr, ndc, b2_r, out_r):
    h2r = (ar0[...] + ar1[...]) * ndr[...][:, None] + b2_r[...]
    h2c = (ac0[...] + ac1[...]) * ndc[...][:, None] + b2_r[...]
    z = lax.dot_general(h2r, h2c, (((1,), (1,)), ((), ())),
                        preferred_element_type=jnp.float32)
    out_r[...] = 0.5 * jnp.tanh(z * 0.5) + 0.5


def _dec_call(a0, a1, nd, b2):
    grid = (NPAD // _DR, NPAD // _DC)
    return pl.pallas_call(
        _dec_body,
        grid=grid,
        in_specs=[
            pl.BlockSpec((_DR, N_HIDDEN), lambda i, j: (i, 0)),
            pl.BlockSpec((_DR, N_HIDDEN), lambda i, j: (i, 0)),
            pl.BlockSpec((_DC, N_HIDDEN), lambda i, j: (j, 0)),
            pl.BlockSpec((_DC, N_HIDDEN), lambda i, j: (j, 0)),
            pl.BlockSpec((_DR,), lambda i, j: (i,)),
            pl.BlockSpec((_DC,), lambda i, j: (j,)),
            pl.BlockSpec((1, N_HIDDEN), lambda i, j: (0, 0)),
        ],
        out_specs=pl.BlockSpec((_DR, _DC), lambda i, j: (i, j)),
        out_shape=jax.ShapeDtypeStruct((N_NODES, N_NODES), jnp.float32),
        compiler_params=pltpu.CompilerParams(
            dimension_semantics=("parallel", "parallel")),
    )(a0, a1, a0, a1, nd, nd, b2)


def kernel(x, edge_index, W1, b1, W2, b2):
    src = edge_index[0].astype(jnp.int32)
    dst = edge_index[1].astype(jnp.int32)
    src3d = src.reshape(NW, NCHUNK, CH)
    dst3d = dst.reshape(NW, NCHUNK, CH)
    src3m = src.reshape(NW, MCHUNK, MCH)
    dst3m = dst.reshape(NW, MCHUNK, MCH)

    outdeg_p, indeg_p = _deg_call(src3d, dst3d)
    ns, nd, hw1 = _enc1_call(outdeg_p[0], outdeg_p[1],
                             indeg_p[0], indeg_p[1], x, W1)
    agg1 = _msg_call(hw1, src3m, dst3m)
    hw2 = _enc2_call(agg1[0], agg1[1], nd, ns, b1.reshape(1, N_HIDDEN), W2)
    agg2 = _msg_call(hw2, src3m, dst3m)
    adj = _dec_call(agg2[0], agg2[1], nd, b2.reshape(1, N_HIDDEN))
    return adj


# speedup vs baseline: 8.6885x; 1.0293x over previous
import functools

import jax
import jax.numpy as jnp
from jax import lax
from jax.experimental import pallas as pl
from jax.experimental.pallas import tpu as pltpu
from jax.experimental.pallas import tpu_sc as plsc

N_NODES = 10000
N_EDGES = 320000
IN_FEATS = 128
N_HIDDEN = 64

NC = 2
NS = 16
NW = NC * NS
MCH = 100
MCHUNK = N_EDGES // NW // MCH
NBUF = 4
NPAD = 10240
SROWS = NPAD // NS

_MESH = functools.partial(
    plsc.VectorSubcoreMesh, core_axis_name="c", subcore_axis_name="s",
    num_cores=NC, num_subcores=NS)

def _deg_body(src_hbm, dst_hbm, outdeg_hbm, indeg_hbm,
              idx_s, idx_d, ones_v, zbuf, degs_sh, degd_sh):
    c = lax.axis_index("c")
    s = lax.axis_index("s")
    wid = c * NS + s
    _Z16 = jnp.zeros((16,), jnp.float32)
    _O16 = jnp.ones((16,), jnp.float32)

    for k in range(112 // 16):
        ones_v[pl.ds(16 * k, 16)] = _O16

    def zrow(i, carry):
        zbuf[pl.ds(i * 16, 16)] = _Z16
        return carry
    lax.fori_loop(0, SROWS // 16, zrow, 0)

    pltpu.sync_copy(zbuf, degs_sh.at[pl.ds(s * SROWS, SROWS)])
    pltpu.sync_copy(zbuf, degd_sh.at[pl.ds(s * SROWS, SROWS)])
    plsc.subcore_barrier()

    pltpu.sync_copy(src_hbm.at[wid], idx_s)
    pltpu.sync_copy(dst_hbm.at[wid], idx_d)

    def body(j, carry):
        pltpu.sync_copy(ones_v.at[pl.ds(0, MCH)], degs_sh.at[idx_s.at[j]],
                        add=True)
        pltpu.sync_copy(ones_v.at[pl.ds(0, MCH)], degd_sh.at[idx_d.at[j]],
                        add=True)
        return carry
    lax.fori_loop(0, MCHUNK, body, 0)

    plsc.subcore_barrier()
    pltpu.sync_copy(degs_sh.at[pl.ds(s * SROWS, SROWS)], zbuf)
    pltpu.sync_copy(zbuf, outdeg_hbm.at[c, pl.ds(s * SROWS, SROWS)])
    pltpu.sync_copy(degd_sh.at[pl.ds(s * SROWS, SROWS)], zbuf)
    pltpu.sync_copy(zbuf, indeg_hbm.at[c, pl.ds(s * SROWS, SROWS)])


_deg_call = pl.kernel(
    _deg_body,
    out_type=(jax.ShapeDtypeStruct((NC, NPAD), jnp.float32),
              jax.ShapeDtypeStruct((NC, NPAD), jnp.float32)),
    mesh=_MESH(),
    scratch_types=[
        pltpu.VMEM((MCHUNK, MCH), jnp.int32),
        pltpu.VMEM((MCHUNK, MCH), jnp.int32),
        pltpu.VMEM((112,), jnp.float32),
        pltpu.VMEM((SROWS,), jnp.float32),
        pltpu.VMEM_SHARED((NPAD,), jnp.float32),
        pltpu.VMEM_SHARED((NPAD,), jnp.float32),
    ],
    compiler_params=pltpu.CompilerParams(use_tc_tiling_on_sc=False),
)


def _msg_body(hw_hbm, src_hbm, dst_hbm, out_hbm,
              idx_s, idx_d, rows, zbuf, acc_sh, gsem, ssem):
    c = lax.axis_index("c")
    s = lax.axis_index("s")
    wid = c * NS + s
    _Z16 = jnp.zeros((16,), jnp.float32)

    def zrow(i, carry):
        for q in range(4):
            zbuf[i, pl.ds(q * 16, 16)] = _Z16
        return carry
    lax.fori_loop(0, 64, zrow, 0)
    for t in range(SROWS // 64):
        pltpu.sync_copy(zbuf, acc_sh.at[pl.ds(s * SROWS + t * 64, 64)])
    plsc.subcore_barrier()

    pltpu.sync_copy(src_hbm.at[wid], idx_s)
    pltpu.sync_copy(dst_hbm.at[wid], idx_d)

    for b in range(NBUF):
        pltpu.async_copy(hw_hbm.at[idx_s.at[b]], rows.at[b], gsem.at[b])

    def body(t, carry):
        j0 = NBUF * t
        for b in range(NBUF):
            pltpu.make_async_copy(hw_hbm.at[idx_s.at[j0 + b]],
                                  rows.at[b], gsem.at[b]).wait()
            pltpu.async_copy(rows.at[b], acc_sh.at[idx_d.at[j0 + b]],
                             ssem.at[b], add=True)
        for b in range(NBUF):
            @pl.when(j0 + b + NBUF < MCHUNK)
            def _():
                pltpu.make_async_copy(rows.at[b],
                                      acc_sh.at[idx_d.at[j0 + b]],
                                      ssem.at[b]).wait()
                pltpu.async_copy(hw_hbm.at[idx_s.at[j0 + b + NBUF]],
                                 rows.at[b], gsem.at[b])
        return carry
    lax.fori_loop(0, MCHUNK // NBUF, body, 0)

    for b in range(NBUF):
        pltpu.make_async_copy(rows.at[b],
                              acc_sh.at[idx_d.at[MCHUNK - NBUF + b]],
                              ssem.at[b]).wait()

    plsc.subcore_barrier()
    for t in range(SROWS // 64):
        pltpu.sync_copy(acc_sh.at[pl.ds(s * SROWS + t * 64, 64)], zbuf)
        pltpu.sync_copy(zbuf, out_hbm.at[c, pl.ds(s * SROWS + t * 64, 64)])


_msg_call = pl.kernel(
    _msg_body,
    out_type=jax.ShapeDtypeStruct((NC, NPAD, N_HIDDEN), jnp.float32),
    mesh=_MESH(),
    scratch_types=[
        pltpu.VMEM((MCHUNK, MCH), jnp.int32),
        pltpu.VMEM((MCHUNK, MCH), jnp.int32),
        pltpu.VMEM((NBUF, MCH, N_HIDDEN), jnp.float32),
        pltpu.VMEM((64, N_HIDDEN), jnp.float32),
        pltpu.VMEM_SHARED((NPAD, N_HIDDEN), jnp.float32),
        pltpu.SemaphoreType.DMA((NBUF,)),
        pltpu.SemaphoreType.DMA((NBUF,)),
    ],
    compiler_params=pltpu.CompilerParams(use_tc_tiling_on_sc=False),
)


_RB = 1024


def _enc1_body(odeg, ideg, x_r, w1_r, ns_r, nd_r, hw1_r):
    ns = lax.rsqrt(jnp.maximum(odeg[0] + odeg[1], 1.0))
    nd = lax.rsqrt(jnp.maximum(ideg[0] + ideg[1], 1.0))
    ns_r[...] = ns
    nd_r[...] = nd
    hw1_r[...] = jnp.dot(x_r[...] * ns[:, None], w1_r[...],
                         preferred_element_type=jnp.float32)


def _enc1_call(odeg, ideg, x, w1):
    grid = (NPAD // _RB,)
    return pl.pallas_call(
        _enc1_body,
        grid=grid,
        in_specs=[
            pl.BlockSpec((NC, _RB), lambda i: (0, i)),
            pl.BlockSpec((NC, _RB), lambda i: (0, i)),
            pl.BlockSpec((_RB, IN_FEATS), lambda i: (i, 0)),
            pl.BlockSpec((IN_FEATS, N_HIDDEN), lambda i: (0, 0)),
        ],
        out_specs=[
            pl.BlockSpec((_RB,), lambda i: (i,)),
            pl.BlockSpec((_RB,), lambda i: (i,)),
            pl.BlockSpec((_RB, N_HIDDEN), lambda i: (i, 0)),
        ],
        out_shape=[
            jax.ShapeDtypeStruct((NPAD,), jnp.float32),
            jax.ShapeDtypeStruct((NPAD,), jnp.float32),
            jax.ShapeDtypeStruct((N_NODES, N_HIDDEN), jnp.float32),
        ],
    )(odeg, ideg, x, w1)


def _enc2_body(a_r, nd_r, ns_r, b1_r, w2_r, hw2_r):
    agg = (a_r[0] + a_r[1]) * nd_r[...][:, None] + b1_r[...]
    h1 = jnp.maximum(agg, 0.0)
    hw2_r[...] = jnp.dot(h1 * ns_r[...][:, None], w2_r[...],
                         preferred_element_type=jnp.float32)


def _enc2_call(a, nd, ns, b1, w2):
    grid = (NPAD // _RB,)
    return pl.pallas_call(
        _enc2_body,
        grid=grid,
        in_specs=[
            pl.BlockSpec((NC, _RB, N_HIDDEN), lambda i: (0, i, 0)),
            pl.BlockSpec((_RB,), lambda i: (i,)),
            pl.BlockSpec((_RB,), lambda i: (i,)),
            pl.BlockSpec((1, N_HIDDEN), lambda i: (0, 0)),
            pl.BlockSpec((N_HIDDEN, N_HIDDEN), lambda i: (0, 0)),
        ],
        out_specs=pl.BlockSpec((_RB, N_HIDDEN), lambda i: (i, 0)),
        out_shape=jax.ShapeDtypeStruct((N_NODES, N_HIDDEN), jnp.float32),
    )(a, nd, ns, b1, w2)


_DR = 512
_DC = 2048


def _dec_body(ar, ac, ndr, ndc, b2_r, out_r):
    h2r = (ar[0] + ar[1]) * ndr[...][:, None] + b2_r[...]
    h2c = (ac[0] + ac[1]) * ndc[...][:, None] + b2_r[...]
    z = lax.dot_general(h2r, h2c, (((1,), (1,)), ((), ())),
                        preferred_element_type=jnp.float32)
    out_r[...] = 0.5 * jnp.tanh(z * 0.5) + 0.5


def _dec_call(a, nd, b2):
    grid = (NPAD // _DR, NPAD // _DC)
    return pl.pallas_call(
        _dec_body,
        grid=grid,
        in_specs=[
            pl.BlockSpec((NC, _DR, N_HIDDEN), lambda i, j: (0, i, 0)),
            pl.BlockSpec((NC, _DC, N_HIDDEN), lambda i, j: (0, j, 0)),
            pl.BlockSpec((_DR,), lambda i, j: (i,)),
            pl.BlockSpec((_DC,), lambda i, j: (j,)),
            pl.BlockSpec((1, N_HIDDEN), lambda i, j: (0, 0)),
        ],
        out_specs=pl.BlockSpec((_DR, _DC), lambda i, j: (i, j)),
        out_shape=jax.ShapeDtypeStruct((N_NODES, N_NODES), jnp.float32),
        compiler_params=pltpu.CompilerParams(
            dimension_semantics=("parallel", "parallel")),
    )(a, a, nd, nd, b2)


def kernel(x, edge_index, W1, b1, W2, b2):
    src = edge_index[0].astype(jnp.int32)
    dst = edge_index[1].astype(jnp.int32)
    src3 = src.reshape(NW, MCHUNK, MCH)
    dst3 = dst.reshape(NW, MCHUNK, MCH)

    outdeg_p, indeg_p = _deg_call(src3, dst3)
    ns, nd, hw1 = _enc1_call(outdeg_p, indeg_p, x, W1)
    agg1 = _msg_call(hw1, src3, dst3)
    hw2 = _enc2_call(agg1, nd, ns, b1.reshape(1, N_HIDDEN), W2)
    agg2 = _msg_call(hw2, src3, dst3)
    adj = _dec_call(agg2, nd, b2.reshape(1, N_HIDDEN))
    return adj


# speedup vs baseline: 10.6907x; 1.2304x over previous
import functools

import jax
import jax.numpy as jnp
from jax import lax
from jax.experimental import pallas as pl
from jax.experimental.pallas import tpu as pltpu
from jax.experimental.pallas import tpu_sc as plsc

N_NODES = 10000
N_EDGES = 320000
IN_FEATS = 128
N_HIDDEN = 64

NC = 2
NS = 16
NW = NC * NS
MCH = 100
MCHUNK = N_EDGES // NW // MCH
NBUF = 4
NPAD = 10240
SROWS = NPAD // NS

_MESH = functools.partial(
    plsc.VectorSubcoreMesh, core_axis_name="c", subcore_axis_name="s",
    num_cores=NC, num_subcores=NS)

def _deg_body(src_hbm, dst_hbm, outdeg_hbm, indeg_hbm,
              idx_s, idx_d, ones_v, zbuf, degs_sh, degd_sh):
    c = lax.axis_index("c")
    s = lax.axis_index("s")
    wid = c * NS + s
    _Z16 = jnp.zeros((16,), jnp.float32)
    _O16 = jnp.ones((16,), jnp.float32)

    for k in range(112 // 16):
        ones_v[pl.ds(16 * k, 16)] = _O16

    def zrow(i, carry):
        zbuf[pl.ds(i * 16, 16)] = _Z16
        return carry
    lax.fori_loop(0, SROWS // 16, zrow, 0)

    pltpu.sync_copy(zbuf, degs_sh.at[pl.ds(s * SROWS, SROWS)])
    pltpu.sync_copy(zbuf, degd_sh.at[pl.ds(s * SROWS, SROWS)])
    plsc.subcore_barrier()

    pltpu.sync_copy(src_hbm.at[wid], idx_s)
    pltpu.sync_copy(dst_hbm.at[wid], idx_d)

    def body(j, carry):
        pltpu.sync_copy(ones_v.at[pl.ds(0, MCH)], degs_sh.at[idx_s.at[j]],
                        add=True)
        pltpu.sync_copy(ones_v.at[pl.ds(0, MCH)], degd_sh.at[idx_d.at[j]],
                        add=True)
        return carry
    lax.fori_loop(0, MCHUNK, body, 0)

    plsc.subcore_barrier()
    pltpu.sync_copy(degs_sh.at[pl.ds(s * SROWS, SROWS)], zbuf)
    pltpu.sync_copy(zbuf, outdeg_hbm.at[c, pl.ds(s * SROWS, SROWS)])
    pltpu.sync_copy(degd_sh.at[pl.ds(s * SROWS, SROWS)], zbuf)
    pltpu.sync_copy(zbuf, indeg_hbm.at[c, pl.ds(s * SROWS, SROWS)])


_deg_call = pl.kernel(
    _deg_body,
    out_type=(jax.ShapeDtypeStruct((NC, NPAD), jnp.float32),
              jax.ShapeDtypeStruct((NC, NPAD), jnp.float32)),
    mesh=_MESH(),
    scratch_types=[
        pltpu.VMEM((MCHUNK, MCH), jnp.int32),
        pltpu.VMEM((MCHUNK, MCH), jnp.int32),
        pltpu.VMEM((112,), jnp.float32),
        pltpu.VMEM((SROWS,), jnp.float32),
        pltpu.VMEM_SHARED((NPAD,), jnp.float32),
        pltpu.VMEM_SHARED((NPAD,), jnp.float32),
    ],
    compiler_params=pltpu.CompilerParams(use_tc_tiling_on_sc=False),
)


def _msg_body(hw_hbm, src_hbm, dst_hbm, out_hbm,
              idx_s, idx_d, rows, zbuf, acc_sh, gsem, ssem):
    c = lax.axis_index("c")
    s = lax.axis_index("s")
    wid = c * NS + s
    _Z16 = jnp.zeros((16,), jnp.float32)

    def zrow(i, carry):
        for q in range(4):
            zbuf[i, pl.ds(q * 16, 16)] = _Z16
        return carry
    lax.fori_loop(0, 64, zrow, 0)
    for t in range(SROWS // 64):
        pltpu.sync_copy(zbuf, acc_sh.at[pl.ds(s * SROWS + t * 64, 64)])
    plsc.subcore_barrier()

    pltpu.sync_copy(src_hbm.at[wid], idx_s)
    pltpu.sync_copy(dst_hbm.at[wid], idx_d)

    for b in range(NBUF):
        pltpu.async_copy(hw_hbm.at[idx_s.at[b]], rows.at[b], gsem.at[b])

    def body(t, carry):
        j0 = NBUF * t
        for b in range(NBUF):
            pltpu.make_async_copy(hw_hbm.at[idx_s.at[j0 + b]],
                                  rows.at[b], gsem.at[b]).wait()
            pltpu.async_copy(rows.at[b], acc_sh.at[idx_d.at[j0 + b]],
                             ssem.at[b], add=True)
        for b in range(NBUF):
            @pl.when(j0 + b + NBUF < MCHUNK)
            def _():
                pltpu.make_async_copy(rows.at[b],
                                      acc_sh.at[idx_d.at[j0 + b]],
                                      ssem.at[b]).wait()
                pltpu.async_copy(hw_hbm.at[idx_s.at[j0 + b + NBUF]],
                                 rows.at[b], gsem.at[b])
        return carry
    lax.fori_loop(0, MCHUNK // NBUF, body, 0)

    for b in range(NBUF):
        pltpu.make_async_copy(rows.at[b],
                              acc_sh.at[idx_d.at[MCHUNK - NBUF + b]],
                              ssem.at[b]).wait()

    plsc.subcore_barrier()
    for t in range(SROWS // 64):
        pltpu.sync_copy(acc_sh.at[pl.ds(s * SROWS + t * 64, 64)], zbuf)
        pltpu.sync_copy(zbuf, out_hbm.at[c, pl.ds(s * SROWS + t * 64, 64)])


_msg_call = pl.kernel(
    _msg_body,
    out_type=jax.ShapeDtypeStruct((NC, NPAD, N_HIDDEN), jnp.float32),
    mesh=_MESH(),
    scratch_types=[
        pltpu.VMEM((MCHUNK, MCH), jnp.int32),
        pltpu.VMEM((MCHUNK, MCH), jnp.int32),
        pltpu.VMEM((NBUF, MCH, N_HIDDEN), jnp.float32),
        pltpu.VMEM((64, N_HIDDEN), jnp.float32),
        pltpu.VMEM_SHARED((NPAD, N_HIDDEN), jnp.float32),
        pltpu.SemaphoreType.DMA((NBUF,)),
        pltpu.SemaphoreType.DMA((NBUF,)),
    ],
    compiler_params=pltpu.CompilerParams(use_tc_tiling_on_sc=False),
)


_RB = 1024


def _enc1_body(odeg, ideg, x_r, w1_r, ns_r, nd_r, hw1_r):
    ns = lax.rsqrt(jnp.maximum(odeg[0] + odeg[1], 1.0))
    nd = lax.rsqrt(jnp.maximum(ideg[0] + ideg[1], 1.0))
    ns_r[...] = ns
    nd_r[...] = nd
    hw1_r[...] = jnp.dot(x_r[...] * ns[:, None], w1_r[...],
                         preferred_element_type=jnp.float32)


def _enc1_call(odeg, ideg, x, w1):
    grid = (NPAD // _RB,)
    return pl.pallas_call(
        _enc1_body,
        grid=grid,
        in_specs=[
            pl.BlockSpec((NC, _RB), lambda i: (0, i)),
            pl.BlockSpec((NC, _RB), lambda i: (0, i)),
            pl.BlockSpec((_RB, IN_FEATS), lambda i: (i, 0)),
            pl.BlockSpec((IN_FEATS, N_HIDDEN), lambda i: (0, 0)),
        ],
        out_specs=[
            pl.BlockSpec((_RB,), lambda i: (i,)),
            pl.BlockSpec((_RB,), lambda i: (i,)),
            pl.BlockSpec((_RB, N_HIDDEN), lambda i: (i, 0)),
        ],
        out_shape=[
            jax.ShapeDtypeStruct((NPAD,), jnp.float32),
            jax.ShapeDtypeStruct((NPAD,), jnp.float32),
            jax.ShapeDtypeStruct((N_NODES, N_HIDDEN), jnp.float32),
        ],
    )(odeg, ideg, x, w1)


def _enc2_body(a_r, nd_r, ns_r, b1_r, w2_r, hw2_r):
    agg = (a_r[0] + a_r[1]) * nd_r[...][:, None] + b1_r[...]
    h1 = jnp.maximum(agg, 0.0)
    hw2_r[...] = jnp.dot(h1 * ns_r[...][:, None], w2_r[...],
                         preferred_element_type=jnp.float32)


def _enc2_call(a, nd, ns, b1, w2):
    grid = (NPAD // _RB,)
    return pl.pallas_call(
        _enc2_body,
        grid=grid,
        in_specs=[
            pl.BlockSpec((NC, _RB, N_HIDDEN), lambda i: (0, i, 0)),
            pl.BlockSpec((_RB,), lambda i: (i,)),
            pl.BlockSpec((_RB,), lambda i: (i,)),
            pl.BlockSpec((1, N_HIDDEN), lambda i: (0, 0)),
            pl.BlockSpec((N_HIDDEN, N_HIDDEN), lambda i: (0, 0)),
        ],
        out_specs=pl.BlockSpec((_RB, N_HIDDEN), lambda i: (i, 0)),
        out_shape=jax.ShapeDtypeStruct((N_NODES, N_HIDDEN), jnp.float32),
    )(a, nd, ns, b1, w2)


_DR = 512
_DC = NPAD


def _dec_body(ar, ac, ndr, ndc, b2_r, out_r):
    h2r = (ar[0] + ar[1]) * ndr[...][:, None] + b2_r[...]
    h2c = (ac[0] + ac[1]) * ndc[...][:, None] + b2_r[...]
    z = lax.dot_general(h2r, h2c, (((1,), (1,)), ((), ())),
                        preferred_element_type=jnp.float32)
    out_r[...] = 0.5 * jnp.tanh(z * 0.5) + 0.5


def _dec_call(a, nd, b2):
    grid = (NPAD // _DR,)
    return pl.pallas_call(
        _dec_body,
        grid=grid,
        in_specs=[
            pl.BlockSpec((NC, _DR, N_HIDDEN), lambda i: (0, i, 0)),
            pl.BlockSpec((NC, _DC, N_HIDDEN), lambda i: (0, 0, 0)),
            pl.BlockSpec((_DR,), lambda i: (i,)),
            pl.BlockSpec((_DC,), lambda i: (0,)),
            pl.BlockSpec((1, N_HIDDEN), lambda i: (0, 0)),
        ],
        out_specs=pl.BlockSpec((_DR, _DC), lambda i: (i, 0)),
        out_shape=jax.ShapeDtypeStruct((N_NODES, N_NODES), jnp.float32),
        compiler_params=pltpu.CompilerParams(
            dimension_semantics=("arbitrary",)),
    )(a, a, nd, nd, b2)


def kernel(x, edge_index, W1, b1, W2, b2):
    src = edge_index[0].astype(jnp.int32)
    dst = edge_index[1].astype(jnp.int32)
    src3 = src.reshape(NW, MCHUNK, MCH)
    dst3 = dst.reshape(NW, MCHUNK, MCH)

    outdeg_p, indeg_p = _deg_call(src3, dst3)
    ns, nd, hw1 = _enc1_call(outdeg_p, indeg_p, x, W1)
    agg1 = _msg_call(hw1, src3, dst3)
    hw2 = _enc2_call(agg1, nd, ns, b1.reshape(1, N_HIDDEN), W2)
    agg2 = _msg_call(hw2, src3, dst3)
    adj = _dec_call(agg2, nd, b2.reshape(1, N_HIDDEN))
    return adj


# speedup vs baseline: 11.0936x; 1.0377x over previous
import functools

import jax
import jax.numpy as jnp
from jax import lax
from jax.experimental import pallas as pl
from jax.experimental.pallas import tpu as pltpu
from jax.experimental.pallas import tpu_sc as plsc

N_NODES = 10000
N_EDGES = 320000
IN_FEATS = 128
N_HIDDEN = 64

NC = 2
NS = 16
NW = NC * NS
MCH = 100
MCHUNK = N_EDGES // NW // MCH
NBUF = 4
NPAD = 10240
SROWS = NPAD // NS

_MESH = functools.partial(
    plsc.VectorSubcoreMesh, core_axis_name="c", subcore_axis_name="s",
    num_cores=NC, num_subcores=NS)

def _deg_body(src_hbm, dst_hbm, outdeg_hbm, indeg_hbm,
              idx_s, idx_d, ones_v, zbuf, degs_sh, degd_sh, ssem, dsem):
    c = lax.axis_index("c")
    s = lax.axis_index("s")
    wid = c * NS + s
    _Z16 = jnp.zeros((16,), jnp.float32)
    _O16 = jnp.ones((16,), jnp.float32)

    for k in range(128 // 16):
        ones_v[pl.ds(16 * k, 16)] = _O16

    def zrow(i, carry):
        zbuf[pl.ds(i * 16, 16)] = _Z16
        return carry
    lax.fori_loop(0, SROWS // 16, zrow, 0)

    pltpu.sync_copy(zbuf, degs_sh.at[pl.ds(s * SROWS, SROWS)])
    pltpu.sync_copy(zbuf, degd_sh.at[pl.ds(s * SROWS, SROWS)])
    plsc.subcore_barrier()

    pltpu.sync_copy(src_hbm.at[wid], idx_s)
    pltpu.sync_copy(dst_hbm.at[wid], idx_d)

    ones = ones_v.at[pl.ds(0, MCH)]

    def body(t, carry):
        j0 = NBUF * t
        for b in range(NBUF):
            @pl.when(t > 0)
            def _():
                pltpu.make_async_copy(
                    ones, degs_sh.at[idx_s.at[j0 + b - NBUF]],
                    ssem.at[b]).wait()
                pltpu.make_async_copy(
                    ones, degd_sh.at[idx_d.at[j0 + b - NBUF]],
                    dsem.at[b]).wait()
            pltpu.async_copy(ones, degs_sh.at[idx_s.at[j0 + b]],
                             ssem.at[b], add=True)
            pltpu.async_copy(ones, degd_sh.at[idx_d.at[j0 + b]],
                             dsem.at[b], add=True)
        return carry
    lax.fori_loop(0, MCHUNK // NBUF, body, 0)
    for b in range(NBUF):
        pltpu.make_async_copy(ones, degs_sh.at[idx_s.at[MCHUNK - NBUF + b]],
                              ssem.at[b]).wait()
        pltpu.make_async_copy(ones, degd_sh.at[idx_d.at[MCHUNK - NBUF + b]],
                              dsem.at[b]).wait()

    plsc.subcore_barrier()
    pltpu.sync_copy(degs_sh.at[pl.ds(s * SROWS, SROWS)], zbuf)
    pltpu.sync_copy(zbuf, outdeg_hbm.at[c, pl.ds(s * SROWS, SROWS)])
    pltpu.sync_copy(degd_sh.at[pl.ds(s * SROWS, SROWS)], zbuf)
    pltpu.sync_copy(zbuf, indeg_hbm.at[c, pl.ds(s * SROWS, SROWS)])


_deg_call = pl.kernel(
    _deg_body,
    out_type=(jax.ShapeDtypeStruct((NC, NPAD), jnp.float32),
              jax.ShapeDtypeStruct((NC, NPAD), jnp.float32)),
    mesh=_MESH(),
    scratch_types=[
        pltpu.VMEM((MCHUNK, MCH), jnp.int32),
        pltpu.VMEM((MCHUNK, MCH), jnp.int32),
        pltpu.VMEM((128,), jnp.float32),
        pltpu.VMEM((SROWS,), jnp.float32),
        pltpu.VMEM_SHARED((NPAD,), jnp.float32),
        pltpu.VMEM_SHARED((NPAD,), jnp.float32),
        pltpu.SemaphoreType.DMA((NBUF,)),
        pltpu.SemaphoreType.DMA((NBUF,)),
    ],
    compiler_params=pltpu.CompilerParams(use_tc_tiling_on_sc=False),
)


def _msg_body(hw_hbm, src_hbm, dst_hbm, out_hbm,
              idx_s, idx_d, rows, zbuf, acc_sh, gsem, ssem):
    c = lax.axis_index("c")
    s = lax.axis_index("s")
    wid = c * NS + s
    _Z16 = jnp.zeros((16,), jnp.float32)

    def zrow(i, carry):
        for q in range(4):
            zbuf[i, pl.ds(q * 16, 16)] = _Z16
        return carry
    lax.fori_loop(0, 64, zrow, 0)
    for t in range(SROWS // 64):
        pltpu.sync_copy(zbuf, acc_sh.at[pl.ds(s * SROWS + t * 64, 64)])
    plsc.subcore_barrier()

    pltpu.sync_copy(src_hbm.at[wid], idx_s)
    pltpu.sync_copy(dst_hbm.at[wid], idx_d)

    for b in range(NBUF):
        pltpu.async_copy(hw_hbm.at[idx_s.at[b]], rows.at[b], gsem.at[b])

    def body(t, carry):
        j0 = NBUF * t
        for b in range(NBUF):
            pltpu.make_async_copy(hw_hbm.at[idx_s.at[j0 + b]],
                                  rows.at[b], gsem.at[b]).wait()
            pltpu.async_copy(rows.at[b], acc_sh.at[idx_d.at[j0 + b]],
                             ssem.at[b], add=True)
        for b in range(NBUF):
            @pl.when(j0 + b + NBUF < MCHUNK)
            def _():
                pltpu.make_async_copy(rows.at[b],
                                      acc_sh.at[idx_d.at[j0 + b]],
                                      ssem.at[b]).wait()
                pltpu.async_copy(hw_hbm.at[idx_s.at[j0 + b + NBUF]],
                                 rows.at[b], gsem.at[b])
        return carry
    lax.fori_loop(0, MCHUNK // NBUF, body, 0)

    for b in range(NBUF):
        pltpu.make_async_copy(rows.at[b],
                              acc_sh.at[idx_d.at[MCHUNK - NBUF + b]],
                              ssem.at[b]).wait()

    plsc.subcore_barrier()
    for t in range(SROWS // 64):
        pltpu.sync_copy(acc_sh.at[pl.ds(s * SROWS + t * 64, 64)], zbuf)
        pltpu.sync_copy(zbuf, out_hbm.at[c, pl.ds(s * SROWS + t * 64, 64)])


_msg_call = pl.kernel(
    _msg_body,
    out_type=jax.ShapeDtypeStruct((NC, NPAD, N_HIDDEN), jnp.float32),
    mesh=_MESH(),
    scratch_types=[
        pltpu.VMEM((MCHUNK, MCH), jnp.int32),
        pltpu.VMEM((MCHUNK, MCH), jnp.int32),
        pltpu.VMEM((NBUF, MCH, N_HIDDEN), jnp.float32),
        pltpu.VMEM((64, N_HIDDEN), jnp.float32),
        pltpu.VMEM_SHARED((NPAD, N_HIDDEN), jnp.float32),
        pltpu.SemaphoreType.DMA((NBUF,)),
        pltpu.SemaphoreType.DMA((NBUF,)),
    ],
    compiler_params=pltpu.CompilerParams(use_tc_tiling_on_sc=False),
)


_RB = 1024


def _enc1_body(odeg, ideg, x_r, w1_r, ns_r, nd_r, hw1_r):
    ns = lax.rsqrt(jnp.maximum(odeg[0] + odeg[1], 1.0))
    nd = lax.rsqrt(jnp.maximum(ideg[0] + ideg[1], 1.0))
    ns_r[...] = ns
    nd_r[...] = nd
    hw1_r[...] = jnp.dot(x_r[...] * ns[:, None], w1_r[...],
                         preferred_element_type=jnp.float32)


def _enc1_call(odeg, ideg, x, w1):
    grid = (NPAD // _RB,)
    return pl.pallas_call(
        _enc1_body,
        grid=grid,
        in_specs=[
            pl.BlockSpec((NC, _RB), lambda i: (0, i)),
            pl.BlockSpec((NC, _RB), lambda i: (0, i)),
            pl.BlockSpec((_RB, IN_FEATS), lambda i: (i, 0)),
            pl.BlockSpec((IN_FEATS, N_HIDDEN), lambda i: (0, 0)),
        ],
        out_specs=[
            pl.BlockSpec((_RB,), lambda i: (i,)),
            pl.BlockSpec((_RB,), lambda i: (i,)),
            pl.BlockSpec((_RB, N_HIDDEN), lambda i: (i, 0)),
        ],
        out_shape=[
            jax.ShapeDtypeStruct((NPAD,), jnp.float32),
            jax.ShapeDtypeStruct((NPAD,), jnp.float32),
            jax.ShapeDtypeStruct((N_NODES, N_HIDDEN), jnp.float32),
        ],
    )(odeg, ideg, x, w1)


def _enc2_body(a_r, nd_r, ns_r, b1_r, w2_r, hw2_r):
    agg = (a_r[0] + a_r[1]) * nd_r[...][:, None] + b1_r[...]
    h1 = jnp.maximum(agg, 0.0)
    hw2_r[...] = jnp.dot(h1 * ns_r[...][:, None], w2_r[...],
                         preferred_element_type=jnp.float32)


def _enc2_call(a, nd, ns, b1, w2):
    grid = (NPAD // _RB,)
    return pl.pallas_call(
        _enc2_body,
        grid=grid,
        in_specs=[
            pl.BlockSpec((NC, _RB, N_HIDDEN), lambda i: (0, i, 0)),
            pl.BlockSpec((_RB,), lambda i: (i,)),
            pl.BlockSpec((_RB,), lambda i: (i,)),
            pl.BlockSpec((1, N_HIDDEN), lambda i: (0, 0)),
            pl.BlockSpec((N_HIDDEN, N_HIDDEN), lambda i: (0, 0)),
        ],
        out_specs=pl.BlockSpec((_RB, N_HIDDEN), lambda i: (i, 0)),
        out_shape=jax.ShapeDtypeStruct((N_NODES, N_HIDDEN), jnp.float32),
    )(a, nd, ns, b1, w2)


_DR = 512
_DC = NPAD


def _dec_body(ar, ac, ndr, ndc, b2_r, out_r):
    h2r = (ar[0] + ar[1]) * ndr[...][:, None] + b2_r[...]
    h2c = (ac[0] + ac[1]) * ndc[...][:, None] + b2_r[...]
    z = lax.dot_general(h2r, h2c, (((1,), (1,)), ((), ())),
                        preferred_element_type=jnp.float32)
    out_r[...] = 0.5 * jnp.tanh(z * 0.5) + 0.5


def _dec_call(a, nd, b2):
    grid = (NPAD // _DR,)
    return pl.pallas_call(
        _dec_body,
        grid=grid,
        in_specs=[
            pl.BlockSpec((NC, _DR, N_HIDDEN), lambda i: (0, i, 0)),
            pl.BlockSpec((NC, _DC, N_HIDDEN), lambda i: (0, 0, 0)),
            pl.BlockSpec((_DR,), lambda i: (i,)),
            pl.BlockSpec((_DC,), lambda i: (0,)),
            pl.BlockSpec((1, N_HIDDEN), lambda i: (0, 0)),
        ],
        out_specs=pl.BlockSpec((_DR, _DC), lambda i: (i, 0)),
        out_shape=jax.ShapeDtypeStruct((N_NODES, N_NODES), jnp.float32),
        compiler_params=pltpu.CompilerParams(
            dimension_semantics=("arbitrary",)),
    )(a, a, nd, nd, b2)


def kernel(x, edge_index, W1, b1, W2, b2):
    src = edge_index[0].astype(jnp.int32)
    dst = edge_index[1].astype(jnp.int32)
    src3 = src.reshape(NW, MCHUNK, MCH)
    dst3 = dst.reshape(NW, MCHUNK, MCH)

    outdeg_p, indeg_p = _deg_call(src3, dst3)
    ns, nd, hw1 = _enc1_call(outdeg_p, indeg_p, x, W1)
    agg1 = _msg_call(hw1, src3, dst3)
    hw2 = _enc2_call(agg1, nd, ns, b1.reshape(1, N_HIDDEN), W2)
    agg2 = _msg_call(hw2, src3, dst3)
    adj = _dec_call(agg2, nd, b2.reshape(1, N_HIDDEN))
    return adj


# speedup vs baseline: 11.2309x; 1.0124x over previous
import functools

import jax
import jax.numpy as jnp
from jax import lax
from jax.experimental import pallas as pl
from jax.experimental.pallas import tpu as pltpu
from jax.experimental.pallas import tpu_sc as plsc

N_NODES = 10000
N_EDGES = 320000
IN_FEATS = 128
N_HIDDEN = 64

NC = 2
NS = 16
NW = NC * NS
MCH = 100
MCHUNK = N_EDGES // NW // MCH
NBUF = 4
NPAD = 10240
SROWS = NPAD // NS

_MESH = functools.partial(
    plsc.VectorSubcoreMesh, core_axis_name="c", subcore_axis_name="s",
    num_cores=NC, num_subcores=NS)

def _deg_body(src_hbm, dst_hbm, outdeg_hbm, indeg_hbm,
              idx_s, idx_d, ones_v, zbuf, degs_sh, degd_sh, ssem, dsem):
    c = lax.axis_index("c")
    s = lax.axis_index("s")
    wid = c * NS + s
    _Z16 = jnp.zeros((16,), jnp.float32)
    _O16 = jnp.ones((16,), jnp.float32)

    for k in range(128 // 16):
        ones_v[pl.ds(16 * k, 16)] = _O16

    def zrow(i, carry):
        zbuf[pl.ds(i * 16, 16)] = _Z16
        return carry
    lax.fori_loop(0, SROWS // 16, zrow, 0)

    pltpu.sync_copy(zbuf, degs_sh.at[pl.ds(s * SROWS, SROWS)])
    pltpu.sync_copy(zbuf, degd_sh.at[pl.ds(s * SROWS, SROWS)])
    plsc.subcore_barrier()

    pltpu.sync_copy(src_hbm.at[wid], idx_s)
    pltpu.sync_copy(dst_hbm.at[wid], idx_d)

    ones = ones_v.at[pl.ds(0, MCH)]

    def body(t, carry):
        j0 = NBUF * t
        for b in range(NBUF):
            @pl.when(t > 0)
            def _():
                pltpu.make_async_copy(
                    ones, degs_sh.at[idx_s.at[j0 + b - NBUF]],
                    ssem.at[b]).wait()
                pltpu.make_async_copy(
                    ones, degd_sh.at[idx_d.at[j0 + b - NBUF]],
                    dsem.at[b]).wait()
            pltpu.async_copy(ones, degs_sh.at[idx_s.at[j0 + b]],
                             ssem.at[b], add=True)
            pltpu.async_copy(ones, degd_sh.at[idx_d.at[j0 + b]],
                             dsem.at[b], add=True)
        return carry
    lax.fori_loop(0, MCHUNK // NBUF, body, 0)
    for b in range(NBUF):
        pltpu.make_async_copy(ones, degs_sh.at[idx_s.at[MCHUNK - NBUF + b]],
                              ssem.at[b]).wait()
        pltpu.make_async_copy(ones, degd_sh.at[idx_d.at[MCHUNK - NBUF + b]],
                              dsem.at[b]).wait()

    plsc.subcore_barrier()
    pltpu.sync_copy(degs_sh.at[pl.ds(s * SROWS, SROWS)], zbuf)
    pltpu.sync_copy(zbuf, outdeg_hbm.at[c, pl.ds(s * SROWS, SROWS)])
    pltpu.sync_copy(degd_sh.at[pl.ds(s * SROWS, SROWS)], zbuf)
    pltpu.sync_copy(zbuf, indeg_hbm.at[c, pl.ds(s * SROWS, SROWS)])


_deg_call = pl.kernel(
    _deg_body,
    out_type=(jax.ShapeDtypeStruct((NC, NPAD), jnp.float32),
              jax.ShapeDtypeStruct((NC, NPAD), jnp.float32)),
    mesh=_MESH(),
    scratch_types=[
        pltpu.VMEM((MCHUNK, MCH), jnp.int32),
        pltpu.VMEM((MCHUNK, MCH), jnp.int32),
        pltpu.VMEM((128,), jnp.float32),
        pltpu.VMEM((SROWS,), jnp.float32),
        pltpu.VMEM_SHARED((NPAD,), jnp.float32),
        pltpu.VMEM_SHARED((NPAD,), jnp.float32),
        pltpu.SemaphoreType.DMA((NBUF,)),
        pltpu.SemaphoreType.DMA((NBUF,)),
    ],
    compiler_params=pltpu.CompilerParams(use_tc_tiling_on_sc=False),
)


def _msg_body(hw_hbm, src_hbm, dst_hbm, out_hbm,
              idx_s, idx_d, rows, zbuf, wb, acc_sh, gsem, ssem, wsem):
    c = lax.axis_index("c")
    s = lax.axis_index("s")
    wid = c * NS + s
    _Z16 = jnp.zeros((16,), jnp.float32)

    def zrow(i, carry):
        for q in range(4):
            zbuf[i, pl.ds(q * 16, 16)] = _Z16
        return carry
    lax.fori_loop(0, 64, zrow, 0)
    for t in range(SROWS // 64):
        pltpu.async_copy(zbuf, acc_sh.at[pl.ds(s * SROWS + t * 64, 64)],
                         wsem.at[t % 2])
    for t in range(SROWS // 64):
        pltpu.make_async_copy(zbuf, acc_sh.at[pl.ds(s * SROWS + t * 64, 64)],
                              wsem.at[t % 2]).wait()
    plsc.subcore_barrier()

    pltpu.sync_copy(src_hbm.at[wid], idx_s)
    pltpu.sync_copy(dst_hbm.at[wid], idx_d)

    for b in range(NBUF):
        pltpu.async_copy(hw_hbm.at[idx_s.at[b]], rows.at[b], gsem.at[b])

    def body(t, carry):
        j0 = NBUF * t
        for b in range(NBUF):
            pltpu.make_async_copy(hw_hbm.at[idx_s.at[j0 + b]],
                                  rows.at[b], gsem.at[b]).wait()
            pltpu.async_copy(rows.at[b], acc_sh.at[idx_d.at[j0 + b]],
                             ssem.at[b], add=True)
        for b in range(NBUF):
            @pl.when(j0 + b + NBUF < MCHUNK)
            def _():
                pltpu.make_async_copy(rows.at[b],
                                      acc_sh.at[idx_d.at[j0 + b]],
                                      ssem.at[b]).wait()
                pltpu.async_copy(hw_hbm.at[idx_s.at[j0 + b + NBUF]],
                                 rows.at[b], gsem.at[b])
        return carry
    lax.fori_loop(0, MCHUNK // NBUF, body, 0)

    for b in range(NBUF):
        pltpu.make_async_copy(rows.at[b],
                              acc_sh.at[idx_d.at[MCHUNK - NBUF + b]],
                              ssem.at[b]).wait()

    plsc.subcore_barrier()
    quarter = SROWS // 4
    for q in range(4):
        b = q % 2
        if q >= 2:
            pltpu.make_async_copy(
                wb.at[b],
                out_hbm.at[c, pl.ds(s * SROWS + (q - 2) * quarter, quarter)],
                wsem.at[b]).wait()
        pltpu.sync_copy(acc_sh.at[pl.ds(s * SROWS + q * quarter, quarter)],
                        wb.at[b])
        pltpu.async_copy(wb.at[b],
                         out_hbm.at[c, pl.ds(s * SROWS + q * quarter, quarter)],
                         wsem.at[b])
    for q in range(2, 4):
        b = q % 2
        pltpu.make_async_copy(
            wb.at[b], out_hbm.at[c, pl.ds(s * SROWS + q * quarter, quarter)],
            wsem.at[b]).wait()


_msg_call = pl.kernel(
    _msg_body,
    out_type=jax.ShapeDtypeStruct((NC, NPAD, N_HIDDEN), jnp.float32),
    mesh=_MESH(),
    scratch_types=[
        pltpu.VMEM((MCHUNK, MCH), jnp.int32),
        pltpu.VMEM((MCHUNK, MCH), jnp.int32),
        pltpu.VMEM((NBUF, MCH, N_HIDDEN), jnp.float32),
        pltpu.VMEM((64, N_HIDDEN), jnp.float32),
        pltpu.VMEM((2, SROWS // 4, N_HIDDEN), jnp.float32),
        pltpu.VMEM_SHARED((NPAD, N_HIDDEN), jnp.float32),
        pltpu.SemaphoreType.DMA((NBUF,)),
        pltpu.SemaphoreType.DMA((NBUF,)),
        pltpu.SemaphoreType.DMA((2,)),
    ],
    compiler_params=pltpu.CompilerParams(use_tc_tiling_on_sc=False),
)


_RB = 1024


def _enc1_body(odeg, ideg, x_r, w1_r, ns_r, nd_r, hw1_r):
    ns = lax.rsqrt(jnp.maximum(odeg[0] + odeg[1], 1.0))
    nd = lax.rsqrt(jnp.maximum(ideg[0] + ideg[1], 1.0))
    ns_r[...] = ns
    nd_r[...] = nd
    hw1_r[...] = jnp.dot(x_r[...] * ns[:, None], w1_r[...],
                         preferred_element_type=jnp.float32)


def _enc1_call(odeg, ideg, x, w1):
    grid = (NPAD // _RB,)
    return pl.pallas_call(
        _enc1_body,
        grid=grid,
        in_specs=[
            pl.BlockSpec((NC, _RB), lambda i: (0, i)),
            pl.BlockSpec((NC, _RB), lambda i: (0, i)),
            pl.BlockSpec((_RB, IN_FEATS), lambda i: (i, 0)),
            pl.BlockSpec((IN_FEATS, N_HIDDEN), lambda i: (0, 0)),
        ],
        out_specs=[
            pl.BlockSpec((_RB,), lambda i: (i,)),
            pl.BlockSpec((_RB,), lambda i: (i,)),
            pl.BlockSpec((_RB, N_HIDDEN), lambda i: (i, 0)),
        ],
        out_shape=[
            jax.ShapeDtypeStruct((NPAD,), jnp.float32),
            jax.ShapeDtypeStruct((NPAD,), jnp.float32),
            jax.ShapeDtypeStruct((N_NODES, N_HIDDEN), jnp.float32),
        ],
    )(odeg, ideg, x, w1)


def _enc2_body(a_r, nd_r, ns_r, b1_r, w2_r, hw2_r):
    agg = (a_r[0] + a_r[1]) * nd_r[...][:, None] + b1_r[...]
    h1 = jnp.maximum(agg, 0.0)
    hw2_r[...] = jnp.dot(h1 * ns_r[...][:, None], w2_r[...],
                         preferred_element_type=jnp.float32)


def _enc2_call(a, nd, ns, b1, w2):
    grid = (NPAD // _RB,)
    return pl.pallas_call(
        _enc2_body,
        grid=grid,
        in_specs=[
            pl.BlockSpec((NC, _RB, N_HIDDEN), lambda i: (0, i, 0)),
            pl.BlockSpec((_RB,), lambda i: (i,)),
            pl.BlockSpec((_RB,), lambda i: (i,)),
            pl.BlockSpec((1, N_HIDDEN), lambda i: (0, 0)),
            pl.BlockSpec((N_HIDDEN, N_HIDDEN), lambda i: (0, 0)),
        ],
        out_specs=pl.BlockSpec((_RB, N_HIDDEN), lambda i: (i, 0)),
        out_shape=jax.ShapeDtypeStruct((N_NODES, N_HIDDEN), jnp.float32),
    )(a, nd, ns, b1, w2)


_DR = 512
_DC = NPAD


def _dec_body(ar, ac, ndr, ndc, b2_r, out_r):
    h2r = (ar[0] + ar[1]) * ndr[...][:, None] + b2_r[...]
    h2c = (ac[0] + ac[1]) * ndc[...][:, None] + b2_r[...]
    z = lax.dot_general(h2r, h2c, (((1,), (1,)), ((), ())),
                        preferred_element_type=jnp.float32)
    out_r[...] = 0.5 * jnp.tanh(z * 0.5) + 0.5


def _dec_call(a, nd, b2):
    grid = (NPAD // _DR,)
    return pl.pallas_call(
        _dec_body,
        grid=grid,
        in_specs=[
            pl.BlockSpec((NC, _DR, N_HIDDEN), lambda i: (0, i, 0)),
            pl.BlockSpec((NC, _DC, N_HIDDEN), lambda i: (0, 0, 0)),
            pl.BlockSpec((_DR,), lambda i: (i,)),
            pl.BlockSpec((_DC,), lambda i: (0,)),
            pl.BlockSpec((1, N_HIDDEN), lambda i: (0, 0)),
        ],
        out_specs=pl.BlockSpec((_DR, _DC), lambda i: (i, 0)),
        out_shape=jax.ShapeDtypeStruct((N_NODES, N_NODES), jnp.float32),
        compiler_params=pltpu.CompilerParams(
            dimension_semantics=("arbitrary",)),
    )(a, a, nd, nd, b2)


def kernel(x, edge_index, W1, b1, W2, b2):
    src = edge_index[0].astype(jnp.int32)
    dst = edge_index[1].astype(jnp.int32)
    src3 = src.reshape(NW, MCHUNK, MCH)
    dst3 = dst.reshape(NW, MCHUNK, MCH)

    outdeg_p, indeg_p = _deg_call(src3, dst3)
    ns, nd, hw1 = _enc1_call(outdeg_p, indeg_p, x, W1)
    agg1 = _msg_call(hw1, src3, dst3)
    hw2 = _enc2_call(agg1, nd, ns, b1.reshape(1, N_HIDDEN), W2)
    agg2 = _msg_call(hw2, src3, dst3)
    adj = _dec_call(agg2, nd, b2.reshape(1, N_HIDDEN))
    return adj


# speedup vs baseline: 11.3233x; 1.0082x over previous
import functools

import jax
import jax.numpy as jnp
from jax import lax
from jax.experimental import pallas as pl
from jax.experimental.pallas import tpu as pltpu
from jax.experimental.pallas import tpu_sc as plsc

N_NODES = 10000
N_EDGES = 320000
IN_FEATS = 128
N_HIDDEN = 64

NC = 2
NS = 16
NW = NC * NS
MCH = 100
MCHUNK = N_EDGES // NW // MCH
NBUF = 5
NPAD = 10240
SROWS = NPAD // NS

_MESH = functools.partial(
    plsc.VectorSubcoreMesh, core_axis_name="c", subcore_axis_name="s",
    num_cores=NC, num_subcores=NS)

def _deg_body(src_hbm, dst_hbm, outdeg_hbm, indeg_hbm,
              idx_s, idx_d, ones_v, zbuf, degs_sh, degd_sh, ssem, dsem):
    c = lax.axis_index("c")
    s = lax.axis_index("s")
    wid = c * NS + s
    _Z16 = jnp.zeros((16,), jnp.float32)
    _O16 = jnp.ones((16,), jnp.float32)

    for k in range(128 // 16):
        ones_v[pl.ds(16 * k, 16)] = _O16

    def zrow(i, carry):
        zbuf[pl.ds(i * 16, 16)] = _Z16
        return carry
    lax.fori_loop(0, SROWS // 16, zrow, 0)

    pltpu.sync_copy(zbuf, degs_sh.at[pl.ds(s * SROWS, SROWS)])
    pltpu.sync_copy(zbuf, degd_sh.at[pl.ds(s * SROWS, SROWS)])
    plsc.subcore_barrier()

    pltpu.sync_copy(src_hbm.at[wid], idx_s)
    pltpu.sync_copy(dst_hbm.at[wid], idx_d)

    ones = ones_v.at[pl.ds(0, MCH)]

    def body(t, carry):
        j0 = NBUF * t
        for b in range(NBUF):
            @pl.when(t > 0)
            def _():
                pltpu.make_async_copy(
                    ones, degs_sh.at[idx_s.at[j0 + b - NBUF]],
                    ssem.at[b]).wait()
                pltpu.make_async_copy(
                    ones, degd_sh.at[idx_d.at[j0 + b - NBUF]],
                    dsem.at[b]).wait()
            pltpu.async_copy(ones, degs_sh.at[idx_s.at[j0 + b]],
                             ssem.at[b], add=True)
            pltpu.async_copy(ones, degd_sh.at[idx_d.at[j0 + b]],
                             dsem.at[b], add=True)
        return carry
    lax.fori_loop(0, MCHUNK // NBUF, body, 0)
    for b in range(NBUF):
        pltpu.make_async_copy(ones, degs_sh.at[idx_s.at[MCHUNK - NBUF + b]],
                              ssem.at[b]).wait()
        pltpu.make_async_copy(ones, degd_sh.at[idx_d.at[MCHUNK - NBUF + b]],
                              dsem.at[b]).wait()

    plsc.subcore_barrier()
    pltpu.sync_copy(degs_sh.at[pl.ds(s * SROWS, SROWS)], zbuf)
    pltpu.sync_copy(zbuf, outdeg_hbm.at[c, pl.ds(s * SROWS, SROWS)])
    pltpu.sync_copy(degd_sh.at[pl.ds(s * SROWS, SROWS)], zbuf)
    pltpu.sync_copy(zbuf, indeg_hbm.at[c, pl.ds(s * SROWS, SROWS)])


_deg_call = pl.kernel(
    _deg_body,
    out_type=(jax.ShapeDtypeStruct((NC, NPAD), jnp.float32),
              jax.ShapeDtypeStruct((NC, NPAD), jnp.float32)),
    mesh=_MESH(),
    scratch_types=[
        pltpu.VMEM((MCHUNK, MCH), jnp.int32),
        pltpu.VMEM((MCHUNK, MCH), jnp.int32),
        pltpu.VMEM((128,), jnp.float32),
        pltpu.VMEM((SROWS,), jnp.float32),
        pltpu.VMEM_SHARED((NPAD,), jnp.float32),
        pltpu.VMEM_SHARED((NPAD,), jnp.float32),
        pltpu.SemaphoreType.DMA((NBUF,)),
        pltpu.SemaphoreType.DMA((NBUF,)),
    ],
    compiler_params=pltpu.CompilerParams(use_tc_tiling_on_sc=False),
)


def _msg_body(hw_hbm, src_hbm, dst_hbm, out_hbm,
              idx_s, idx_d, rows, zbuf, wb, acc_sh, gsem, ssem, wsem):
    c = lax.axis_index("c")
    s = lax.axis_index("s")
    wid = c * NS + s
    _Z16 = jnp.zeros((16,), jnp.float32)

    def zrow(i, carry):
        for q in range(4):
            zbuf[i, pl.ds(q * 16, 16)] = _Z16
        return carry
    lax.fori_loop(0, 64, zrow, 0)
    for t in range(SROWS // 64):
        pltpu.async_copy(zbuf, acc_sh.at[pl.ds(s * SROWS + t * 64, 64)],
                         wsem.at[t % 2])
    for t in range(SROWS // 64):
        pltpu.make_async_copy(zbuf, acc_sh.at[pl.ds(s * SROWS + t * 64, 64)],
                              wsem.at[t % 2]).wait()
    plsc.subcore_barrier()

    pltpu.sync_copy(src_hbm.at[wid], idx_s)
    pltpu.sync_copy(dst_hbm.at[wid], idx_d)

    for b in range(NBUF):
        pltpu.async_copy(hw_hbm.at[idx_s.at[b]], rows.at[b], gsem.at[b])

    def body(t, carry):
        j0 = NBUF * t
        for b in range(NBUF):
            pltpu.make_async_copy(hw_hbm.at[idx_s.at[j0 + b]],
                                  rows.at[b], gsem.at[b]).wait()
            pltpu.async_copy(rows.at[b], acc_sh.at[idx_d.at[j0 + b]],
                             ssem.at[b], add=True)
        for b in range(NBUF):
            @pl.when(j0 + b + NBUF < MCHUNK)
            def _():
                pltpu.make_async_copy(rows.at[b],
                                      acc_sh.at[idx_d.at[j0 + b]],
                                      ssem.at[b]).wait()
                pltpu.async_copy(hw_hbm.at[idx_s.at[j0 + b + NBUF]],
                                 rows.at[b], gsem.at[b])
        return carry
    lax.fori_loop(0, MCHUNK // NBUF, body, 0)

    for b in range(NBUF):
        pltpu.make_async_copy(rows.at[b],
                              acc_sh.at[idx_d.at[MCHUNK - NBUF + b]],
                              ssem.at[b]).wait()

    plsc.subcore_barrier()
    quarter = SROWS // 4
    for q in range(4):
        b = q % 2
        if q >= 2:
            pltpu.make_async_copy(
                wb.at[b],
                out_hbm.at[c, pl.ds(s * SROWS + (q - 2) * quarter, quarter)],
                wsem.at[b]).wait()
        pltpu.sync_copy(acc_sh.at[pl.ds(s * SROWS + q * quarter, quarter)],
                        wb.at[b])
        pltpu.async_copy(wb.at[b],
                         out_hbm.at[c, pl.ds(s * SROWS + q * quarter, quarter)],
                         wsem.at[b])
    for q in range(2, 4):
        b = q % 2
        pltpu.make_async_copy(
            wb.at[b], out_hbm.at[c, pl.ds(s * SROWS + q * quarter, quarter)],
            wsem.at[b]).wait()


_msg_call = pl.kernel(
    _msg_body,
    out_type=jax.ShapeDtypeStruct((NC, NPAD, N_HIDDEN), jnp.float32),
    mesh=_MESH(),
    scratch_types=[
        pltpu.VMEM((MCHUNK, MCH), jnp.int32),
        pltpu.VMEM((MCHUNK, MCH), jnp.int32),
        pltpu.VMEM((NBUF, MCH, N_HIDDEN), jnp.float32),
        pltpu.VMEM((64, N_HIDDEN), jnp.float32),
        pltpu.VMEM((2, SROWS // 4, N_HIDDEN), jnp.float32),
        pltpu.VMEM_SHARED((NPAD, N_HIDDEN), jnp.float32),
        pltpu.SemaphoreType.DMA((NBUF,)),
        pltpu.SemaphoreType.DMA((NBUF,)),
        pltpu.SemaphoreType.DMA((2,)),
    ],
    compiler_params=pltpu.CompilerParams(use_tc_tiling_on_sc=False),
)


_RB = 1024


def _enc1_body(odeg, ideg, x_r, w1_r, ns_r, nd_r, hw1_r):
    ns = lax.rsqrt(jnp.maximum(odeg[0] + odeg[1], 1.0))
    nd = lax.rsqrt(jnp.maximum(ideg[0] + ideg[1], 1.0))
    ns_r[...] = ns
    nd_r[...] = nd
    hw1_r[...] = jnp.dot(x_r[...] * ns[:, None], w1_r[...],
                         preferred_element_type=jnp.float32)


def _enc1_call(odeg, ideg, x, w1):
    grid = (NPAD // _RB,)
    return pl.pallas_call(
        _enc1_body,
        grid=grid,
        in_specs=[
            pl.BlockSpec((NC, _RB), lambda i: (0, i)),
            pl.BlockSpec((NC, _RB), lambda i: (0, i)),
            pl.BlockSpec((_RB, IN_FEATS), lambda i: (i, 0)),
            pl.BlockSpec((IN_FEATS, N_HIDDEN), lambda i: (0, 0)),
        ],
        out_specs=[
            pl.BlockSpec((_RB,), lambda i: (i,)),
            pl.BlockSpec((_RB,), lambda i: (i,)),
            pl.BlockSpec((_RB, N_HIDDEN), lambda i: (i, 0)),
        ],
        out_shape=[
            jax.ShapeDtypeStruct((NPAD,), jnp.float32),
            jax.ShapeDtypeStruct((NPAD,), jnp.float32),
            jax.ShapeDtypeStruct((N_NODES, N_HIDDEN), jnp.float32),
        ],
    )(odeg, ideg, x, w1)


def _enc2_body(a_r, nd_r, ns_r, b1_r, w2_r, hw2_r):
    agg = (a_r[0] + a_r[1]) * nd_r[...][:, None] + b1_r[...]
    h1 = jnp.maximum(agg, 0.0)
    hw2_r[...] = jnp.dot(h1 * ns_r[...][:, None], w2_r[...],
                         preferred_element_type=jnp.float32)


def _enc2_call(a, nd, ns, b1, w2):
    grid = (NPAD // _RB,)
    return pl.pallas_call(
        _enc2_body,
        grid=grid,
        in_specs=[
            pl.BlockSpec((NC, _RB, N_HIDDEN), lambda i: (0, i, 0)),
            pl.BlockSpec((_RB,), lambda i: (i,)),
            pl.BlockSpec((_RB,), lambda i: (i,)),
            pl.BlockSpec((1, N_HIDDEN), lambda i: (0, 0)),
            pl.BlockSpec((N_HIDDEN, N_HIDDEN), lambda i: (0, 0)),
        ],
        out_specs=pl.BlockSpec((_RB, N_HIDDEN), lambda i: (i, 0)),
        out_shape=jax.ShapeDtypeStruct((N_NODES, N_HIDDEN), jnp.float32),
    )(a, nd, ns, b1, w2)


_DR = 512
_DC = NPAD


def _dec_body(ar, ac, ndr, ndc, b2_r, out_r):
    h2r = (ar[0] + ar[1]) * ndr[...][:, None] + b2_r[...]
    h2c = (ac[0] + ac[1]) * ndc[...][:, None] + b2_r[...]
    z = lax.dot_general(h2r, h2c, (((1,), (1,)), ((), ())),
                        preferred_element_type=jnp.float32)
    out_r[...] = 0.5 * jnp.tanh(z * 0.5) + 0.5


def _dec_call(a, nd, b2):
    grid = (NPAD // _DR,)
    return pl.pallas_call(
        _dec_body,
        grid=grid,
        in_specs=[
            pl.BlockSpec((NC, _DR, N_HIDDEN), lambda i: (0, i, 0)),
            pl.BlockSpec((NC, _DC, N_HIDDEN), lambda i: (0, 0, 0)),
            pl.BlockSpec((_DR,), lambda i: (i,)),
            pl.BlockSpec((_DC,), lambda i: (0,)),
            pl.BlockSpec((1, N_HIDDEN), lambda i: (0, 0)),
        ],
        out_specs=pl.BlockSpec((_DR, _DC), lambda i: (i, 0)),
        out_shape=jax.ShapeDtypeStruct((N_NODES, N_NODES), jnp.float32),
        compiler_params=pltpu.CompilerParams(
            dimension_semantics=("arbitrary",)),
    )(a, a, nd, nd, b2)


def kernel(x, edge_index, W1, b1, W2, b2):
    src = edge_index[0].astype(jnp.int32)
    dst = edge_index[1].astype(jnp.int32)
    src3 = src.reshape(NW, MCHUNK, MCH)
    dst3 = dst.reshape(NW, MCHUNK, MCH)

    outdeg_p, indeg_p = _deg_call(src3, dst3)
    ns, nd, hw1 = _enc1_call(outdeg_p, indeg_p, x, W1)
    agg1 = _msg_call(hw1, src3, dst3)
    hw2 = _enc2_call(agg1, nd, ns, b1.reshape(1, N_HIDDEN), W2)
    agg2 = _msg_call(hw2, src3, dst3)
    adj = _dec_call(agg2, nd, b2.reshape(1, N_HIDDEN))
    return adj
